# Initial kernel scaffold; baseline (speedup 1.0000x reference)
import jax
import jax.numpy as jnp
from jax.experimental import pallas as pl


def kernel(x, e_conv1_w, e_conv1_b, e_conv2_w, e_conv2_b, e_conv3_w, e_conv3_b, e_res_w1, e_res_w2, e_conv4_w, e_conv4_b, codebook, d_convT1_w, d_convT1_b, d_res_w1, d_res_w2, d_convT2_w, d_convT2_b, d_convT3_w, d_convT3_b):
    raise NotImplementedError("write your pallas kernel here")


# speedup vs baseline: 1.2055x; 1.2055x over previous
import jax
import jax.numpy as jnp
from jax import lax
from jax.experimental import pallas as pl
from jax.experimental.pallas import tpu as pltpu


def _conv2d(x, w, b=None, stride=1, padding=0):
    out = jax.lax.conv_general_dilated(
        x, w, (stride, stride), ((padding, padding), (padding, padding)),
        dimension_numbers=('NCHW', 'OIHW', 'NCHW'))
    if b is not None:
        out = out + b[None, :, None, None]
    return out


def _convT2d(x, w, b=None, stride=2, padding=1):
    k = w.shape[2]
    wf = jnp.transpose(w, (1, 0, 2, 3))[:, :, ::-1, ::-1]
    pad = k - 1 - padding
    out = jax.lax.conv_general_dilated(
        x, wf, (1, 1), ((pad, pad), (pad, pad)),
        lhs_dilation=(stride, stride),
        dimension_numbers=('NCHW', 'OIHW', 'NCHW'))
    if b is not None:
        out = out + b[None, :, None, None]
    return out


def _res_stack(x, w1, w2):
    for _ in range(2):
        h = _conv2d(jax.nn.relu(x), w1, None, 1, 1)
        h = _conv2d(jax.nn.relu(h), w2, None, 1, 0)
        x = x + h
    return jax.nn.relu(x)


_K = 8192
_D = 32
_BR = 128


def _vq_body(fnorm_ref, cnorm_ref, flat_ref, cb_ref, q_ref, counts_ref):
    i = pl.program_id(0)
    flat = flat_ref[...]
    cb = cb_ref[...]
    mm = lax.dot_general(flat, cb, (((1,), (1,)), ((), ())),
                         preferred_element_type=jnp.float32)
    dist = (fnorm_ref[...] + cnorm_ref[...]) - 2.0 * mm
    m = jnp.min(dist, axis=1, keepdims=True)
    iota = lax.broadcasted_iota(jnp.int32, (_BR, _K), 1)
    idx = jnp.min(jnp.where(dist == m, iota, _K), axis=1)
    onehot = (iota == idx[:, None]).astype(jnp.float32)
    q_ref[...] = lax.dot_general(onehot, cb, (((1,), (0,)), ((), ())),
                                 preferred_element_type=jnp.float32)

    @pl.when(i == 0)
    def _init():
        counts_ref[...] = jnp.zeros_like(counts_ref)

    counts_ref[...] += jnp.sum(onehot, axis=0, keepdims=True)


def _vq(fnorm, cnorm, flat, cb):
    n = flat.shape[0]
    grid = n // _BR
    return pl.pallas_call(
        _vq_body,
        grid=(grid,),
        in_specs=[
            pl.BlockSpec((_BR, 1), lambda i: (i, 0)),
            pl.BlockSpec((1, _K), lambda i: (0, 0)),
            pl.BlockSpec((_BR, _D), lambda i: (i, 0)),
            pl.BlockSpec((_K, _D), lambda i: (0, 0)),
        ],
        out_specs=[
            pl.BlockSpec((_BR, _D), lambda i: (i, 0)),
            pl.BlockSpec((1, _K), lambda i: (0, 0)),
        ],
        out_shape=[
            jax.ShapeDtypeStruct((n, _D), jnp.float32),
            jax.ShapeDtypeStruct((1, _K), jnp.float32),
        ],
    )(fnorm, cnorm, flat, cb)


def kernel(x, e_conv1_w, e_conv1_b, e_conv2_w, e_conv2_b, e_conv3_w, e_conv3_b,
           e_res_w1, e_res_w2, e_conv4_w, e_conv4_b, codebook,
           d_convT1_w, d_convT1_b, d_res_w1, d_res_w2,
           d_convT2_w, d_convT2_b, d_convT3_w, d_convT3_b):
    h = jax.nn.relu(_conv2d(x, e_conv1_w, e_conv1_b, 2, 1))
    h = jax.nn.relu(_conv2d(h, e_conv2_w, e_conv2_b, 2, 1))
    h = _conv2d(h, e_conv3_w, e_conv3_b, 1, 1)
    h = _res_stack(h, e_res_w1, e_res_w2)
    z = _conv2d(h, e_conv4_w, e_conv4_b, 1, 1)
    zi = jnp.transpose(z, (0, 2, 3, 1))
    flat = zi.reshape(-1, codebook.shape[1])
    fnorm = jnp.sum(flat ** 2, axis=1, keepdims=True)
    cnorm = jnp.sum(codebook ** 2, axis=1)[None, :]
    q_flat, counts = _vq(fnorm, cnorm, flat, codebook)
    quantized = q_flat.reshape(zi.shape)
    e_latent_loss = jnp.mean((jax.lax.stop_gradient(quantized) - zi) ** 2)
    q_latent_loss = jnp.mean((quantized - jax.lax.stop_gradient(zi)) ** 2)
    vq_loss = q_latent_loss + 0.25 * e_latent_loss
    quantized_st = zi + jax.lax.stop_gradient(quantized - zi)
    avg_probs = counts[0] / flat.shape[0]
    perplexity = jnp.exp(-jnp.sum(avg_probs * jnp.log(avg_probs + 1e-10)))
    quantized_out = jnp.transpose(quantized_st, (0, 3, 1, 2))
    h = _convT2d(z, d_convT1_w, d_convT1_b, 1, 1)
    h = _res_stack(h, d_res_w1, d_res_w2)
    h = jax.nn.relu(_convT2d(h, d_convT2_w, d_convT2_b, 2, 1))
    x_recon = _convT2d(h, d_convT3_w, d_convT3_b, 2, 1)
    return (x_recon, vq_loss, perplexity, quantized_out)
